# baseline (speedup 1.0000x reference)
import jax
import jax.numpy as jnp
from jax.experimental import pallas as pl
from jax.experimental.pallas import tpu as pltpu

CODEBOOK_SIZE = 128
EMBEDDING_DIM = 32
BATCH = 256
SEQ = 576

BATCH_PER_STEP = 8
TILE_ROWS = BATCH_PER_STEP * SEQ
GRID = BATCH // BATCH_PER_STEP


def _vq_body(lat_ref, cb_ref, policy_ref, quant_ref, cbset_ref):
    lat = lat_ref[...].reshape(TILE_ROWS, EMBEDDING_DIM)
    cb = cb_ref[...]
    cb_norm = jnp.sum(cb * cb, axis=1)
    lat_norm = jnp.sum(lat * lat, axis=1, keepdims=True)
    scores = lat_norm + cb_norm[None, :] - 2.0 * jnp.dot(
        lat, cb.T, preferred_element_type=jnp.float32
    )
    smin = jnp.min(scores, axis=1, keepdims=True)
    code_iota = jax.lax.broadcasted_iota(
        jnp.int32, (TILE_ROWS, CODEBOOK_SIZE), 1
    )
    idx = jnp.min(
        jnp.where(scores == smin, code_iota, CODEBOOK_SIZE), axis=1
    )
    onehot = (idx[:, None] == code_iota).astype(jnp.float32)
    q = jnp.dot(onehot, cb, preferred_element_type=jnp.float32)
    shape3 = (BATCH_PER_STEP, SEQ, EMBEDDING_DIM)
    quant_ref[...] = q.reshape(shape3)
    policy_ref[...] = (lat + (q - lat)).reshape(shape3)
    cbset_ref[...] = jnp.broadcast_to(
        cb[None], (BATCH_PER_STEP, CODEBOOK_SIZE, EMBEDDING_DIM)
    )


def kernel(latent, codebook):
    return pl.pallas_call(
        _vq_body,
        grid=(GRID,),
        in_specs=[
            pl.BlockSpec(
                (BATCH_PER_STEP, SEQ, EMBEDDING_DIM), lambda i: (i, 0, 0)
            ),
            pl.BlockSpec((CODEBOOK_SIZE, EMBEDDING_DIM), lambda i: (0, 0)),
        ],
        out_specs=[
            pl.BlockSpec(
                (BATCH_PER_STEP, SEQ, EMBEDDING_DIM), lambda i: (i, 0, 0)
            ),
            pl.BlockSpec(
                (BATCH_PER_STEP, SEQ, EMBEDDING_DIM), lambda i: (i, 0, 0)
            ),
            pl.BlockSpec(
                (BATCH_PER_STEP, CODEBOOK_SIZE, EMBEDDING_DIM), lambda i: (i, 0, 0)
            ),
        ],
        out_shape=[
            jax.ShapeDtypeStruct((BATCH, SEQ, EMBEDDING_DIM), jnp.float32),
            jax.ShapeDtypeStruct((BATCH, SEQ, EMBEDDING_DIM), jnp.float32),
            jax.ShapeDtypeStruct((BATCH, CODEBOOK_SIZE, EMBEDDING_DIM), jnp.float32),
        ],
        compiler_params=pltpu.CompilerParams(
            dimension_semantics=("parallel",),
        ),
    )(latent, codebook)


# speedup vs baseline: 1.0313x; 1.0313x over previous
import jax
import jax.numpy as jnp
from jax.experimental import pallas as pl
from jax.experimental.pallas import tpu as pltpu

CODEBOOK_SIZE = 128
EMBEDDING_DIM = 32
BATCH = 256
SEQ = 576

BATCH_PER_STEP = 16
TILE_ROWS = BATCH_PER_STEP * SEQ
GRID = BATCH // BATCH_PER_STEP


def _vq_body(lat_ref, cb_ref, policy_ref, quant_ref, cbset_ref):
    lat = lat_ref[...].reshape(TILE_ROWS, EMBEDDING_DIM)
    cb = cb_ref[...]
    cb_norm = jnp.sum(cb * cb, axis=1)
    lat_norm = jnp.sum(lat * lat, axis=1, keepdims=True)
    scores = lat_norm + cb_norm[None, :] - 2.0 * jnp.dot(
        lat, cb.T, preferred_element_type=jnp.float32
    )
    smin = jnp.min(scores, axis=1, keepdims=True)
    code_iota = jax.lax.broadcasted_iota(
        jnp.int32, (TILE_ROWS, CODEBOOK_SIZE), 1
    )
    idx = jnp.min(
        jnp.where(scores == smin, code_iota, CODEBOOK_SIZE), axis=1
    )
    onehot = (idx[:, None] == code_iota).astype(jnp.float32)
    q = jnp.dot(onehot, cb, preferred_element_type=jnp.float32)
    shape3 = (BATCH_PER_STEP, SEQ, EMBEDDING_DIM)
    quant_ref[...] = q.reshape(shape3)
    policy_ref[...] = (lat + (q - lat)).reshape(shape3)
    cbset_ref[...] = jnp.broadcast_to(
        cb[None], (BATCH_PER_STEP, CODEBOOK_SIZE, EMBEDDING_DIM)
    )


def kernel(latent, codebook):
    return pl.pallas_call(
        _vq_body,
        grid=(GRID,),
        in_specs=[
            pl.BlockSpec(
                (BATCH_PER_STEP, SEQ, EMBEDDING_DIM), lambda i: (i, 0, 0)
            ),
            pl.BlockSpec((CODEBOOK_SIZE, EMBEDDING_DIM), lambda i: (0, 0)),
        ],
        out_specs=[
            pl.BlockSpec(
                (BATCH_PER_STEP, SEQ, EMBEDDING_DIM), lambda i: (i, 0, 0)
            ),
            pl.BlockSpec(
                (BATCH_PER_STEP, SEQ, EMBEDDING_DIM), lambda i: (i, 0, 0)
            ),
            pl.BlockSpec(
                (BATCH_PER_STEP, CODEBOOK_SIZE, EMBEDDING_DIM), lambda i: (i, 0, 0)
            ),
        ],
        out_shape=[
            jax.ShapeDtypeStruct((BATCH, SEQ, EMBEDDING_DIM), jnp.float32),
            jax.ShapeDtypeStruct((BATCH, SEQ, EMBEDDING_DIM), jnp.float32),
            jax.ShapeDtypeStruct((BATCH, CODEBOOK_SIZE, EMBEDDING_DIM), jnp.float32),
        ],
        compiler_params=pltpu.CompilerParams(
            dimension_semantics=("parallel",),
        ),
    )(latent, codebook)


# speedup vs baseline: 1.7982x; 1.7437x over previous
import jax
import jax.numpy as jnp
from jax.experimental import pallas as pl
from jax.experimental.pallas import tpu as pltpu

CODEBOOK_SIZE = 128
EMBEDDING_DIM = 32
BATCH = 256
SEQ = 576
PACK = 128 // EMBEDDING_DIM
SEQP = SEQ // PACK

BATCH_PER_STEP = 8
ROWS = BATCH_PER_STEP * SEQP
GRID = BATCH // BATCH_PER_STEP


def _vq_body(lat_ref, cb_ref, cbt_ref, cbflat_ref,
             policy_ref, quant_ref, cbset_ref):
    lat = lat_ref[...].reshape(ROWS, 128)
    cb = cb_ref[...]
    cbt = cbt_ref[...]
    cb_norm = jnp.sum(cb * cb, axis=1)[None, :]
    b_full = jnp.concatenate([cbt, cbt, cbt, cbt], axis=0)
    w_full = jnp.concatenate([cb, cb, cb, cb], axis=1)
    sub32 = jax.lax.broadcasted_iota(jnp.int32, (128, 128), 0) // EMBEDDING_DIM
    lane32 = jax.lax.broadcasted_iota(jnp.int32, (128, 128), 1) // EMBEDDING_DIM
    iota = jax.lax.broadcasted_iota(jnp.int32, (ROWS, 128), 1).astype(
        jnp.float32
    )
    latsq = lat * lat
    q = jnp.zeros((ROWS, 128), jnp.float32)
    for j in range(PACK):
        b_j = jnp.where(sub32 == j, b_full, 0.0)
        mm_j = jnp.dot(lat, b_j, preferred_element_type=jnp.float32)
        ln_j = jnp.sum(
            latsq[:, j * EMBEDDING_DIM:(j + 1) * EMBEDDING_DIM],
            axis=1, keepdims=True,
        )
        scores_j = (ln_j + cb_norm) - 2.0 * mm_j
        smin_j = jnp.min(scores_j, axis=1, keepdims=True)
        idx_j = jnp.min(
            jnp.where(scores_j == smin_j, iota, float(CODEBOOK_SIZE)),
            axis=1, keepdims=True,
        )
        onehot_j = (iota == idx_j).astype(jnp.float32)
        w_j = jnp.where(lane32 == j, w_full, 0.0)
        q = q + jnp.dot(onehot_j, w_j, preferred_element_type=jnp.float32)
    shape3 = (BATCH_PER_STEP, SEQP, 128)
    quant_ref[...] = q.reshape(shape3)
    policy_ref[...] = (lat + (q - lat)).reshape(shape3)
    cbset_ref[...] = jnp.broadcast_to(
        cbflat_ref[...][None], (BATCH_PER_STEP, EMBEDDING_DIM, 128)
    )


def kernel(latent, codebook):
    latp = latent.reshape(BATCH, SEQP, 128)
    cbt = jnp.swapaxes(codebook, 0, 1)
    cbflat = codebook.reshape(EMBEDDING_DIM, 128)
    policy, quant, cbset = pl.pallas_call(
        _vq_body,
        grid=(GRID,),
        in_specs=[
            pl.BlockSpec((BATCH_PER_STEP, SEQP, 128), lambda i: (i, 0, 0)),
            pl.BlockSpec((CODEBOOK_SIZE, EMBEDDING_DIM), lambda i: (0, 0)),
            pl.BlockSpec((EMBEDDING_DIM, 128), lambda i: (0, 0)),
            pl.BlockSpec((EMBEDDING_DIM, 128), lambda i: (0, 0)),
        ],
        out_specs=[
            pl.BlockSpec((BATCH_PER_STEP, SEQP, 128), lambda i: (i, 0, 0)),
            pl.BlockSpec((BATCH_PER_STEP, SEQP, 128), lambda i: (i, 0, 0)),
            pl.BlockSpec(
                (BATCH_PER_STEP, EMBEDDING_DIM, 128), lambda i: (i, 0, 0)
            ),
        ],
        out_shape=[
            jax.ShapeDtypeStruct((BATCH, SEQP, 128), jnp.float32),
            jax.ShapeDtypeStruct((BATCH, SEQP, 128), jnp.float32),
            jax.ShapeDtypeStruct((BATCH, EMBEDDING_DIM, 128), jnp.float32),
        ],
        compiler_params=pltpu.CompilerParams(
            dimension_semantics=("parallel",),
        ),
    )(latp, codebook, cbt, cbflat)
    shape3 = (BATCH, SEQ, EMBEDDING_DIM)
    return (
        policy.reshape(shape3),
        quant.reshape(shape3),
        cbset.reshape(BATCH, CODEBOOK_SIZE, EMBEDDING_DIM),
    )
